# baseline (speedup 1.0000x reference)
import functools

import jax
import jax.numpy as jnp
from jax import lax
from jax.experimental import pallas as pl
from jax.experimental.pallas import tpu as pltpu
from jax.experimental.pallas import tpu_sc as plsc

TOK = 8192
HID = 2048
E = 8

MBLK = 256
MBLKS = TOK // MBLK
NL = MBLKS + E - 1
NLP = 40

GATE_BLK = 512

NC = 2
NS = 16
NW = NC * NS
RPW = TOK // NW
CH = 16
NCH = RPW // CH


def _gate_route_body(x_ref, wg_ref, pos_ref, starts_ref, ends_ref, bid_ref,
                     eid_ref, valid_ref, t1_acc):
    i = pl.program_id(0)
    x = x_ref[...]
    wg = wg_ref[...]
    logits = lax.dot_general(x, wg, (((1,), (1,)), ((), ())),
                             preferred_element_type=jnp.float32)
    mx = jnp.max(logits, axis=1, keepdims=True)
    eidx = lax.broadcasted_iota(jnp.int32, logits.shape, 1)
    cand = jnp.where(logits == mx, eidx, E)
    t1_acc[pl.ds(i * GATE_BLK, GATE_BLK), :] = jnp.min(cand, axis=1,
                                                       keepdims=True)

    @pl.when(i == TOK // GATE_BLK - 1)
    def _():
        _route_compute(t1_acc, pos_ref, starts_ref, ends_ref, bid_ref,
                       eid_ref, valid_ref)


def _route_compute(top1_ref, pos_ref, starts_ref, ends_ref, bid_ref, eid_ref,
                   valid_ref):
    t1 = top1_ref[...]
    eids = lax.broadcasted_iota(jnp.int32, (TOK, E), 1)
    onehot = (t1 == eids).astype(jnp.int32)

    csum = onehot
    k = 1
    while k < TOK:
        csum = csum + jnp.concatenate(
            [jnp.zeros((k, E), jnp.int32), csum[:TOK - k, :]], axis=0)
        k *= 2
    counts = csum[TOK - 1:TOK, :]

    incl = counts
    k = 1
    while k < E:
        incl = incl + jnp.concatenate(
            [jnp.zeros((1, k), jnp.int32), incl[:, :E - k]], axis=1)
        k *= 2
    starts = incl - counts
    ends = incl

    rank_excl = csum - onehot
    pos = jnp.sum(onehot * (starts + rank_excl), axis=1, keepdims=True)
    pos_ref[...] = pos
    starts_ref[...] = starts
    ends_ref[...] = ends

    nonempty = counts > 0
    first_blk = starts // MBLK
    last_blk = (ends - 1) // MBLK
    tiles = jnp.where(nonempty, last_blk - first_blk + 1, 0)
    ct = tiles
    k = 1
    while k < E:
        ct = ct + jnp.concatenate(
            [jnp.zeros((1, k), jnp.int32), ct[:, :E - k]], axis=1)
        k *= 2
    ct_excl = ct - tiles
    total = ct[:, E - 1:E]

    l_ids = lax.broadcasted_iota(jnp.int32, (NLP, 1), 0)
    ct_b = jnp.broadcast_to(ct, (NLP, E))
    g = jnp.sum((ct_b <= l_ids).astype(jnp.int32), axis=1, keepdims=True)
    valid = l_ids < total
    gc = jnp.minimum(g, E - 1)
    onehot_g = (gc == lax.broadcasted_iota(jnp.int32, (NLP, E), 1)).astype(
        jnp.int32)
    fb = jnp.sum(onehot_g * jnp.broadcast_to(first_blk, (NLP, E)), axis=1,
                 keepdims=True)
    cte = jnp.sum(onehot_g * jnp.broadcast_to(ct_excl, (NLP, E)), axis=1,
                  keepdims=True)
    bid = fb + (l_ids - cte)
    bid_ref[...] = jnp.where(valid, bid, MBLKS - 1)
    eid_ref[...] = jnp.where(valid, gc, E - 1)
    valid_ref[...] = valid.astype(jnp.int32)


def _gate_route(tokens, Wg):
    nsteps = TOK // GATE_BLK
    return pl.pallas_call(
        _gate_route_body,
        grid=(nsteps,),
        in_specs=[
            pl.BlockSpec((GATE_BLK, HID), lambda i: (i, 0)),
            pl.BlockSpec((E, HID), lambda i: (0, 0)),
        ],
        out_specs=(
            pl.BlockSpec((TOK, 1), lambda i: (0, 0)),
            pl.BlockSpec((1, E), lambda i: (0, 0)),
            pl.BlockSpec((1, E), lambda i: (0, 0)),
            pl.BlockSpec((NLP, 1), lambda i: (0, 0)),
            pl.BlockSpec((NLP, 1), lambda i: (0, 0)),
            pl.BlockSpec((NLP, 1), lambda i: (0, 0)),
        ),
        out_shape=(
            jax.ShapeDtypeStruct((TOK, 1), jnp.int32),
            jax.ShapeDtypeStruct((1, E), jnp.int32),
            jax.ShapeDtypeStruct((1, E), jnp.int32),
            jax.ShapeDtypeStruct((NLP, 1), jnp.int32),
            jax.ShapeDtypeStruct((NLP, 1), jnp.int32),
            jax.ShapeDtypeStruct((NLP, 1), jnp.int32),
        ),
        scratch_shapes=[pltpu.VMEM((TOK, 1), jnp.int32)],
        compiler_params=pltpu.CompilerParams(
            dimension_semantics=("arbitrary",)),
    )(tokens, Wg)


def _sc_mesh():
    return plsc.VectorSubcoreMesh(core_axis_name="c", subcore_axis_name="s")


NBUF = 3


def _sc_scratch():
    return ([pltpu.VMEM((CH,), jnp.int32) for _ in range(NBUF)]
            + [pltpu.VMEM((CH, HID), jnp.float32) for _ in range(NBUF)]
            + [pltpu.SemaphoreType.DMA for _ in range(2 * NBUF)])


def _sc_scatter_rows(src, pos):
    @functools.partial(
        pl.kernel,
        out_type=jax.ShapeDtypeStruct((TOK, HID), jnp.float32),
        mesh=_sc_mesh(),
        scratch_types=_sc_scratch(),
    )
    def body(src_hbm, pos_hbm, out_hbm, *scratch):
        idx = scratch[:NBUF]
        rows = scratch[NBUF:2 * NBUF]
        lsem = scratch[2 * NBUF:3 * NBUF]
        ssem = scratch[3 * NBUF:4 * NBUF]
        wid = lax.axis_index("s") * NC + lax.axis_index("c")
        base = wid * RPW

        def load(c, b):
            off = base + c * CH
            d1 = pltpu.async_copy(pos_hbm.at[pl.ds(off, CH)], idx[b], lsem[b])
            d2 = pltpu.async_copy(src_hbm.at[pl.ds(off, CH)], rows[b], lsem[b])
            return (d1, d2)

        loads = [None] * NBUF
        scats = [None] * NBUF
        for c in range(min(NBUF, NCH)):
            loads[c] = load(c, c)
        for c in range(NCH):
            b = c % NBUF
            loads[b][0].wait()
            loads[b][1].wait()
            scats[b] = pltpu.async_copy(rows[b], out_hbm.at[idx[b]], ssem[b])
            nc = c + NBUF
            if nc < NCH:
                scats[b].wait()
                scats[b] = None
                loads[b] = load(nc, b)
        for b in range(NBUF):
            if scats[b] is not None:
                scats[b].wait()

    return body(src, pos)


def _sc_gather_rows(src, pos):
    @functools.partial(
        pl.kernel,
        out_type=jax.ShapeDtypeStruct((TOK, HID), jnp.float32),
        mesh=_sc_mesh(),
        scratch_types=_sc_scratch(),
    )
    def body(src_hbm, pos_hbm, out_hbm, *scratch):
        idx = scratch[:NBUF]
        rows = scratch[NBUF:2 * NBUF]
        gsem = scratch[2 * NBUF:3 * NBUF]
        stsem = scratch[3 * NBUF:4 * NBUF]
        wid = lax.axis_index("s") * NC + lax.axis_index("c")
        base = wid * RPW

        def gather(c, b):
            off = base + c * CH
            pltpu.sync_copy(pos_hbm.at[pl.ds(off, CH)], idx[b])
            return pltpu.async_copy(src_hbm.at[idx[b]], rows[b], gsem[b])

        gats = [None] * NBUF
        stores = [None] * NBUF
        for c in range(min(NBUF, NCH)):
            gats[c] = gather(c, c)
        for c in range(NCH):
            b = c % NBUF
            gats[b].wait()
            off = base + c * CH
            stores[b] = pltpu.async_copy(rows[b], out_hbm.at[pl.ds(off, CH)],
                                         stsem[b])
            nc = c + NBUF
            if nc < NCH:
                stores[b].wait()
                stores[b] = None
                gats[b] = gather(nc, b)
        for b in range(NBUF):
            if stores[b] is not None:
                stores[b].wait()

    return body(src, pos)


def _moe_body(bid_ref, eid_ref, valid_ref, s_ref, e_ref,
              xs_ref, we_ref, be_ref, wc_ref, out_ref):
    l = pl.program_id(0)

    @pl.when(valid_ref[l] == 1)
    def _():
        x = xs_ref[...]
        w = we_ref[0]
        h = lax.dot_general(x, w, (((1,), (1,)), ((), ())),
                            preferred_element_type=jnp.float32)
        h = h + be_ref[0]
        h = 0.5 * h * (1.0 + lax.erf(h * 0.7071067811865476))
        g = eid_ref[l]
        rows = bid_ref[l] * MBLK + lax.broadcasted_iota(
            jnp.int32, (MBLK, 1), 0)
        mask = (rows >= s_ref[g]) & (rows < e_ref[g])
        hm = jnp.where(mask, h, 0.0)
        y = lax.dot_general(hm, wc_ref[...], (((1,), (1,)), ((), ())),
                            preferred_element_type=jnp.float32)
        out_ref[...] = jnp.where(mask, y, out_ref[...])


def _moe(bid, eid, valid, starts, ends, xs, We, be, Wc):
    grid_spec = pltpu.PrefetchScalarGridSpec(
        num_scalar_prefetch=5,
        grid=(NL,),
        in_specs=[
            pl.BlockSpec((MBLK, HID), lambda l, b, e, v, s, en: (b[l], 0)),
            pl.BlockSpec((1, HID, HID), lambda l, b, e, v, s, en: (e[l], 0, 0)),
            pl.BlockSpec((1, 1, HID), lambda l, b, e, v, s, en: (e[l], 0, 0)),
            pl.BlockSpec((HID, HID), lambda l, b, e, v, s, en: (0, 0)),
        ],
        out_specs=pl.BlockSpec((MBLK, HID), lambda l, b, e, v, s, en: (b[l], 0)),
    )
    return pl.pallas_call(
        _moe_body,
        grid_spec=grid_spec,
        out_shape=jax.ShapeDtypeStruct((TOK, HID), jnp.float32),
        compiler_params=pltpu.CompilerParams(
            dimension_semantics=("arbitrary",),
            vmem_limit_bytes=100 * 1024 * 1024),
    )(bid, eid, valid, starts, ends, xs, We, be.reshape(E, 1, HID), Wc)


def kernel(tokens, Wg, We, be, Wc):
    pos2, starts2, ends2, bid2, eid2, valid2 = _gate_route(tokens, Wg)
    pos = pos2.reshape(TOK)
    bid = bid2.reshape(NLP)
    eid = eid2.reshape(NLP)
    valid = valid2.reshape(NLP)
    starts = starts2.reshape(E)
    ends = ends2.reshape(E)
    xs = _sc_scatter_rows(tokens, pos)
    ys = _moe(bid, eid, valid, starts, ends, xs, We, be, Wc)
    return _sc_gather_rows(ys, pos)


# speedup vs baseline: 1.0366x; 1.0366x over previous
import functools

import jax
import jax.numpy as jnp
from jax import lax
from jax.experimental import pallas as pl
from jax.experimental.pallas import tpu as pltpu
from jax.experimental.pallas import tpu_sc as plsc

TOK = 8192
HID = 2048
E = 8

MBLK = 256
MBLKS = TOK // MBLK
NL = MBLKS + E - 1
NLP = 40

GATE_BLK = 512

NC = 2
NS = 16
NW = NC * NS
RPW = TOK // NW
CH = 16
NCH = RPW // CH


def _gate_route_body(x_ref, wg_ref, pos_ref, starts_ref, ends_ref, bid_ref,
                     eid_ref, valid_ref, xbf_ref, t1_acc):
    i = pl.program_id(0)
    x = x_ref[...]
    u = lax.bitcast_convert_type(x, jnp.uint32)
    lsb = (u >> 16) & jnp.uint32(1)
    r = (u + jnp.uint32(0x7FFF) + lsb) >> 16
    a = r[:, :HID // 2]
    b = r[:, HID // 2:]
    xbf_ref[...] = lax.bitcast_convert_type(a | (b << 16), jnp.int32)
    wg = wg_ref[...]
    logits = lax.dot_general(x, wg, (((1,), (1,)), ((), ())),
                             preferred_element_type=jnp.float32)
    mx = jnp.max(logits, axis=1, keepdims=True)
    eidx = lax.broadcasted_iota(jnp.int32, logits.shape, 1)
    cand = jnp.where(logits == mx, eidx, E)
    t1_acc[pl.ds(i * GATE_BLK, GATE_BLK), :] = jnp.min(cand, axis=1,
                                                       keepdims=True)

    @pl.when(i == TOK // GATE_BLK - 1)
    def _():
        _route_compute(t1_acc, pos_ref, starts_ref, ends_ref, bid_ref,
                       eid_ref, valid_ref)


def _route_compute(top1_ref, pos_ref, starts_ref, ends_ref, bid_ref, eid_ref,
                   valid_ref):
    t1 = top1_ref[...]
    eids = lax.broadcasted_iota(jnp.int32, (TOK, E), 1)
    onehot = (t1 == eids).astype(jnp.int32)

    csum = onehot
    k = 1
    while k < TOK:
        csum = csum + jnp.concatenate(
            [jnp.zeros((k, E), jnp.int32), csum[:TOK - k, :]], axis=0)
        k *= 2
    counts = csum[TOK - 1:TOK, :]

    incl = counts
    k = 1
    while k < E:
        incl = incl + jnp.concatenate(
            [jnp.zeros((1, k), jnp.int32), incl[:, :E - k]], axis=1)
        k *= 2
    starts = incl - counts
    ends = incl

    rank_excl = csum - onehot
    pos = jnp.sum(onehot * (starts + rank_excl), axis=1, keepdims=True)
    pos_ref[...] = pos
    starts_ref[...] = starts
    ends_ref[...] = ends

    nonempty = counts > 0
    first_blk = starts // MBLK
    last_blk = (ends - 1) // MBLK
    tiles = jnp.where(nonempty, last_blk - first_blk + 1, 0)
    ct = tiles
    k = 1
    while k < E:
        ct = ct + jnp.concatenate(
            [jnp.zeros((1, k), jnp.int32), ct[:, :E - k]], axis=1)
        k *= 2
    ct_excl = ct - tiles
    total = ct[:, E - 1:E]

    l_ids = lax.broadcasted_iota(jnp.int32, (NLP, 1), 0)
    ct_b = jnp.broadcast_to(ct, (NLP, E))
    g = jnp.sum((ct_b <= l_ids).astype(jnp.int32), axis=1, keepdims=True)
    valid = l_ids < total
    gc = jnp.minimum(g, E - 1)
    onehot_g = (gc == lax.broadcasted_iota(jnp.int32, (NLP, E), 1)).astype(
        jnp.int32)
    fb = jnp.sum(onehot_g * jnp.broadcast_to(first_blk, (NLP, E)), axis=1,
                 keepdims=True)
    cte = jnp.sum(onehot_g * jnp.broadcast_to(ct_excl, (NLP, E)), axis=1,
                  keepdims=True)
    bid = fb + (l_ids - cte)
    bid_ref[...] = jnp.where(valid, bid, MBLKS - 1)
    eid_ref[...] = jnp.where(valid, gc, E - 1)
    valid_ref[...] = valid.astype(jnp.int32)


def _gate_route(tokens, Wg):
    nsteps = TOK // GATE_BLK
    return pl.pallas_call(
        _gate_route_body,
        grid=(nsteps,),
        in_specs=[
            pl.BlockSpec((GATE_BLK, HID), lambda i: (i, 0)),
            pl.BlockSpec((E, HID), lambda i: (0, 0)),
        ],
        out_specs=(
            pl.BlockSpec((TOK, 1), lambda i: (0, 0)),
            pl.BlockSpec((1, E), lambda i: (0, 0)),
            pl.BlockSpec((1, E), lambda i: (0, 0)),
            pl.BlockSpec((NLP, 1), lambda i: (0, 0)),
            pl.BlockSpec((NLP, 1), lambda i: (0, 0)),
            pl.BlockSpec((NLP, 1), lambda i: (0, 0)),
            pl.BlockSpec((GATE_BLK, HID // 2), lambda i: (i, 0)),
        ),
        out_shape=(
            jax.ShapeDtypeStruct((TOK, 1), jnp.int32),
            jax.ShapeDtypeStruct((1, E), jnp.int32),
            jax.ShapeDtypeStruct((1, E), jnp.int32),
            jax.ShapeDtypeStruct((NLP, 1), jnp.int32),
            jax.ShapeDtypeStruct((NLP, 1), jnp.int32),
            jax.ShapeDtypeStruct((NLP, 1), jnp.int32),
            jax.ShapeDtypeStruct((TOK, HID // 2), jnp.int32),
        ),
        scratch_shapes=[pltpu.VMEM((TOK, 1), jnp.int32)],
        compiler_params=pltpu.CompilerParams(
            dimension_semantics=("arbitrary",)),
    )(tokens, Wg)


def _sc_mesh():
    return plsc.VectorSubcoreMesh(core_axis_name="c", subcore_axis_name="s")


NBUF = 3


def _sc_scratch(width, dtype):
    return ([pltpu.VMEM((CH,), jnp.int32) for _ in range(NBUF)]
            + [pltpu.VMEM((CH, width), dtype) for _ in range(NBUF)]
            + [pltpu.SemaphoreType.DMA for _ in range(2 * NBUF)])


def _sc_scatter_rows(src, pos):
    @functools.partial(
        pl.kernel,
        out_type=jax.ShapeDtypeStruct((TOK, HID // 2), jnp.int32),
        mesh=_sc_mesh(),
        scratch_types=_sc_scratch(HID // 2, jnp.int32),
    )
    def body(src_hbm, pos_hbm, out_hbm, *scratch):
        idx = scratch[:NBUF]
        rows = scratch[NBUF:2 * NBUF]
        lsem = scratch[2 * NBUF:3 * NBUF]
        ssem = scratch[3 * NBUF:4 * NBUF]
        wid = lax.axis_index("s") * NC + lax.axis_index("c")
        base = wid * RPW

        def load(c, b):
            off = base + c * CH
            d1 = pltpu.async_copy(pos_hbm.at[pl.ds(off, CH)], idx[b], lsem[b])
            d2 = pltpu.async_copy(src_hbm.at[pl.ds(off, CH)], rows[b], lsem[b])
            return (d1, d2)

        loads = [None] * NBUF
        scats = [None] * NBUF
        for c in range(min(NBUF, NCH)):
            loads[c] = load(c, c)
        for c in range(NCH):
            b = c % NBUF
            loads[b][0].wait()
            loads[b][1].wait()
            scats[b] = pltpu.async_copy(rows[b], out_hbm.at[idx[b]], ssem[b])
            nc = c + NBUF
            if nc < NCH:
                scats[b].wait()
                scats[b] = None
                loads[b] = load(nc, b)
        for b in range(NBUF):
            if scats[b] is not None:
                scats[b].wait()

    return body(src, pos)


def _sc_gather_rows(src, pos):
    @functools.partial(
        pl.kernel,
        out_type=jax.ShapeDtypeStruct((TOK, HID), jnp.float32),
        mesh=_sc_mesh(),
        scratch_types=_sc_scratch(HID, jnp.float32),
    )
    def body(src_hbm, pos_hbm, out_hbm, *scratch):
        idx = scratch[:NBUF]
        rows = scratch[NBUF:2 * NBUF]
        gsem = scratch[2 * NBUF:3 * NBUF]
        stsem = scratch[3 * NBUF:4 * NBUF]
        wid = lax.axis_index("s") * NC + lax.axis_index("c")
        base = wid * RPW

        def gather(c, b):
            off = base + c * CH
            pltpu.sync_copy(pos_hbm.at[pl.ds(off, CH)], idx[b])
            return pltpu.async_copy(src_hbm.at[idx[b]], rows[b], gsem[b])

        gats = [None] * NBUF
        stores = [None] * NBUF
        for c in range(min(NBUF, NCH)):
            gats[c] = gather(c, c)
        for c in range(NCH):
            b = c % NBUF
            gats[b].wait()
            off = base + c * CH
            stores[b] = pltpu.async_copy(rows[b], out_hbm.at[pl.ds(off, CH)],
                                         stsem[b])
            nc = c + NBUF
            if nc < NCH:
                stores[b].wait()
                stores[b] = None
                gats[b] = gather(nc, b)
        for b in range(NBUF):
            if stores[b] is not None:
                stores[b].wait()

    return body(src, pos)


def _moe_body(bid_ref, eid_ref, valid_ref, s_ref, e_ref,
              xs_ref, we_ref, be_ref, wc_ref, out_ref):
    l = pl.program_id(0)

    @pl.when(valid_ref[l] == 1)
    def _():
        v = lax.bitcast_convert_type(xs_ref[...], jnp.uint32)
        lo = lax.bitcast_convert_type(v << 16, jnp.float32)
        hi = lax.bitcast_convert_type(v & jnp.uint32(0xFFFF0000), jnp.float32)
        x = jnp.concatenate([lo, hi], axis=1)
        w = we_ref[0]
        h = lax.dot_general(x, w, (((1,), (1,)), ((), ())),
                            preferred_element_type=jnp.float32)
        h = h + be_ref[0]
        h = 0.5 * h * (1.0 + lax.erf(h * 0.7071067811865476))
        g = eid_ref[l]
        rows = bid_ref[l] * MBLK + lax.broadcasted_iota(
            jnp.int32, (MBLK, 1), 0)
        mask = (rows >= s_ref[g]) & (rows < e_ref[g])
        hm = jnp.where(mask, h, 0.0)
        y = lax.dot_general(hm, wc_ref[...], (((1,), (1,)), ((), ())),
                            preferred_element_type=jnp.float32)
        out_ref[...] = jnp.where(mask, y, out_ref[...])


def _moe(bid, eid, valid, starts, ends, xs, We, be, Wc):
    grid_spec = pltpu.PrefetchScalarGridSpec(
        num_scalar_prefetch=5,
        grid=(NL,),
        in_specs=[
            pl.BlockSpec((MBLK, HID // 2),
                         lambda l, b, e, v, s, en: (b[l], 0)),
            pl.BlockSpec((1, HID, HID), lambda l, b, e, v, s, en: (e[l], 0, 0)),
            pl.BlockSpec((1, 1, HID), lambda l, b, e, v, s, en: (e[l], 0, 0)),
            pl.BlockSpec((HID, HID), lambda l, b, e, v, s, en: (0, 0)),
        ],
        out_specs=pl.BlockSpec((MBLK, HID), lambda l, b, e, v, s, en: (b[l], 0)),
    )
    return pl.pallas_call(
        _moe_body,
        grid_spec=grid_spec,
        out_shape=jax.ShapeDtypeStruct((TOK, HID), jnp.float32),
        compiler_params=pltpu.CompilerParams(
            dimension_semantics=("arbitrary",),
            vmem_limit_bytes=100 * 1024 * 1024),
    )(bid, eid, valid, starts, ends, xs, We, be.reshape(E, 1, HID), Wc)


def kernel(tokens, Wg, We, be, Wc):
    pos2, starts2, ends2, bid2, eid2, valid2, tokens_bf = _gate_route(tokens,
                                                                      Wg)
    pos = pos2.reshape(TOK)
    bid = bid2.reshape(NLP)
    eid = eid2.reshape(NLP)
    valid = valid2.reshape(NLP)
    starts = starts2.reshape(E)
    ends = ends2.reshape(E)
    xs = _sc_scatter_rows(tokens_bf, pos)
    ys = _moe(bid, eid, valid, starts, ends, xs, We, be, Wc)
    return _sc_gather_rows(ys, pos)
